# Initial kernel scaffold; baseline (speedup 1.0000x reference)
import functools

import jax
import jax.numpy as jnp
from jax.experimental import pallas as pl

N = 4096
SEQ = 6
DIM = 3
K = 16
NTYPES = 9
EMB = 16
RADIUS = 0.015
BLOCK_R = 256


def _knn_kernel(psf_ref, post_ref, types_ref, embed_ref,
                node_ref, send_ref, recv_ref, dx_ref, dy_ref, dz_ref, dd_ref):
    i = pl.program_id(0)
    R = BLOCK_R

    psf = psf_ref[...]
    mrx = psf[:, 15:16]
    mry = psf[:, 16:17]
    mrz = psf[:, 17:18]

    px = post_ref[0:1, :]
    py = post_ref[1:2, :]
    pz = post_ref[2:3, :]

    sq_all = (px * px + py * py) + pz * pz
    sq_row = (mrx * mrx + mry * mry) + mrz * mrz

    dot = (mrx * px + mry * py) + mrz * pz
    d2 = (sq_row + sq_all) - 2.0 * dot

    col = jax.lax.broadcasted_iota(jnp.int32, (R, N), 1)
    inf = jnp.float32(jnp.inf)

    idx_cols = []
    sx_cols = []
    sy_cols = []
    sz_cols = []
    for _ in range(K):
        m = jnp.min(d2, axis=1, keepdims=True)
        eq = d2 == m
        idx = jnp.min(jnp.where(eq, col, jnp.int32(N)), axis=1,
                      keepdims=True)
        sel = col == idx
        d2 = jnp.where(sel, inf, d2)
        sx = jnp.sum(jnp.where(sel, px, 0.0), axis=1, keepdims=True)
        sy = jnp.sum(jnp.where(sel, py, 0.0), axis=1, keepdims=True)
        sz = jnp.sum(jnp.where(sel, pz, 0.0), axis=1, keepdims=True)
        idx_cols.append(idx)
        sx_cols.append(sx)
        sy_cols.append(sy)
        sz_cols.append(sz)

    send_ref[...] = jnp.concatenate(idx_cols, axis=1)
    row = jax.lax.broadcasted_iota(jnp.int32, (R, K), 0)
    recv_ref[...] = row + i * R

    sxk = jnp.concatenate(sx_cols, axis=1)
    syk = jnp.concatenate(sy_cols, axis=1)
    szk = jnp.concatenate(sz_cols, axis=1)
    inv_r = jnp.float32(RADIUS)
    dx = (sxk - mrx) / inv_r
    dy = (syk - mry) / inv_r
    dz = (szk - mrz) / inv_r
    dd = jnp.sqrt(((dx * dx + dy * dy) + dz * dz) + 1e-12)
    dx_ref[...] = dx
    dy_ref[...] = dy
    dz_ref[...] = dz
    dd_ref[...] = dd

    vel = psf[:, 3:18] - psf[:, 0:15]
    mrp = psf[:, 15:18]
    lower = mrp - jnp.float32(0.0)
    upper = jnp.float32(1.0) - mrp
    bnd = jnp.concatenate([lower, upper], axis=1) / inv_r
    bnd = jnp.clip(bnd, -1.0, 1.0)
    types = types_ref[...]
    emb = jnp.zeros((R, EMB), dtype=jnp.float32)
    for t in range(NTYPES):
        rowv = embed_ref[t:t + 1, :]
        emb = jnp.where(types == t, rowv, emb)
    node_ref[...] = jnp.concatenate([vel, bnd, emb], axis=1)


def kernel(position_sequence, particle_types, embed_table):
    psf = position_sequence.reshape(N, SEQ * DIM)
    mrp = position_sequence[:, SEQ - 1, :]
    post = jnp.zeros((8, N), jnp.float32).at[0:3, :].set(mrp.T)
    types = particle_types.astype(jnp.int32).reshape(N, 1)

    grid = (N // BLOCK_R,)
    out_shapes = (
        jax.ShapeDtypeStruct((N, 37), jnp.float32),
        jax.ShapeDtypeStruct((N, K), jnp.int32),
        jax.ShapeDtypeStruct((N, K), jnp.int32),
        jax.ShapeDtypeStruct((N, K), jnp.float32),
        jax.ShapeDtypeStruct((N, K), jnp.float32),
        jax.ShapeDtypeStruct((N, K), jnp.float32),
        jax.ShapeDtypeStruct((N, K), jnp.float32),
    )
    in_specs = [
        pl.BlockSpec((BLOCK_R, SEQ * DIM), lambda i: (i, 0)),
        pl.BlockSpec((8, N), lambda i: (0, 0)),
        pl.BlockSpec((BLOCK_R, 1), lambda i: (i, 0)),
        pl.BlockSpec((NTYPES, EMB), lambda i: (0, 0)),
    ]
    out_specs = (
        pl.BlockSpec((BLOCK_R, 37), lambda i: (i, 0)),
        pl.BlockSpec((BLOCK_R, K), lambda i: (i, 0)),
        pl.BlockSpec((BLOCK_R, K), lambda i: (i, 0)),
        pl.BlockSpec((BLOCK_R, K), lambda i: (i, 0)),
        pl.BlockSpec((BLOCK_R, K), lambda i: (i, 0)),
        pl.BlockSpec((BLOCK_R, K), lambda i: (i, 0)),
        pl.BlockSpec((BLOCK_R, K), lambda i: (i, 0)),
    )
    node, send, recv, dx, dy, dz, dd = pl.pallas_call(
        _knn_kernel,
        grid=grid,
        in_specs=in_specs,
        out_specs=out_specs,
        out_shape=out_shapes,
    )(psf, post, types, embed_table)

    edge_index = jnp.stack([send.reshape(-1), recv.reshape(-1)], axis=0)
    edge_features = jnp.stack([dx, dy, dz, dd], axis=-1).reshape(N * K, 4)
    return node, edge_index, edge_features


# speedup vs baseline: 4.6875x; 4.6875x over previous
import functools

import jax
import jax.numpy as jnp
from jax.experimental import pallas as pl

N = 4096
SEQ = 6
DIM = 3
K = 16
NTYPES = 9
EMB = 16
RADIUS = 0.015
BLOCK_R = 256


def _knn_kernel(psf_ref, post_ref, types_ref, embed_ref,
                node_ref, send_ref, recv_ref, dx_ref, dy_ref, dz_ref, dd_ref):
    i = pl.program_id(0)
    R = BLOCK_R

    psf = psf_ref[...]
    mrx = psf[:, 15:16]
    mry = psf[:, 16:17]
    mrz = psf[:, 17:18]

    px = post_ref[0:1, :]
    py = post_ref[1:2, :]
    pz = post_ref[2:3, :]

    sq_all = (px * px + py * py) + pz * pz
    sq_row = (mrx * mrx + mry * mry) + mrz * mrz

    bf = jnp.bfloat16
    f32 = jnp.float32
    qx = px.astype(bf).astype(f32)
    qy = py.astype(bf).astype(f32)
    qz = pz.astype(bf).astype(f32)
    bx = mrx.astype(bf).astype(f32)
    by = mry.astype(bf).astype(f32)
    bz = mrz.astype(bf).astype(f32)
    dot = (bx * qx + by * qy) + bz * qz
    d2 = (sq_row + sq_all) - 2.0 * dot

    col = jax.lax.broadcasted_iota(jnp.int32, (R, N), 1)
    inf = jnp.float32(jnp.inf)

    idx_cols = []
    sx_cols = []
    sy_cols = []
    sz_cols = []
    for _ in range(K):
        m = jnp.min(d2, axis=1, keepdims=True)
        eq = d2 == m
        idx = jnp.min(jnp.where(eq, col, jnp.int32(N)), axis=1,
                      keepdims=True)
        sel = col == idx
        d2 = jnp.where(sel, inf, d2)
        sx = jnp.sum(jnp.where(sel, px, 0.0), axis=1, keepdims=True)
        sy = jnp.sum(jnp.where(sel, py, 0.0), axis=1, keepdims=True)
        sz = jnp.sum(jnp.where(sel, pz, 0.0), axis=1, keepdims=True)
        idx_cols.append(idx)
        sx_cols.append(sx)
        sy_cols.append(sy)
        sz_cols.append(sz)

    send_ref[...] = jnp.concatenate(idx_cols, axis=1)
    row = jax.lax.broadcasted_iota(jnp.int32, (R, K), 0)
    recv_ref[...] = row + i * R

    sxk = jnp.concatenate(sx_cols, axis=1)
    syk = jnp.concatenate(sy_cols, axis=1)
    szk = jnp.concatenate(sz_cols, axis=1)
    inv_r = jnp.float32(RADIUS)
    dx = (sxk - mrx) / inv_r
    dy = (syk - mry) / inv_r
    dz = (szk - mrz) / inv_r
    dd = jnp.sqrt(((dx * dx + dy * dy) + dz * dz) + 1e-12)
    dx_ref[...] = dx
    dy_ref[...] = dy
    dz_ref[...] = dz
    dd_ref[...] = dd

    vel = psf[:, 3:18] - psf[:, 0:15]
    mrp = psf[:, 15:18]
    lower = mrp - jnp.float32(0.0)
    upper = jnp.float32(1.0) - mrp
    bnd = jnp.concatenate([lower, upper], axis=1) / inv_r
    bnd = jnp.clip(bnd, -1.0, 1.0)
    types = types_ref[...]
    emb = jnp.zeros((R, EMB), dtype=jnp.float32)
    for t in range(NTYPES):
        rowv = embed_ref[t:t + 1, :]
        emb = jnp.where(types == t, rowv, emb)
    node_ref[...] = jnp.concatenate([vel, bnd, emb], axis=1)


def kernel(position_sequence, particle_types, embed_table):
    psf = position_sequence.reshape(N, SEQ * DIM)
    mrp = position_sequence[:, SEQ - 1, :]
    post = jnp.zeros((8, N), jnp.float32).at[0:3, :].set(mrp.T)
    types = particle_types.astype(jnp.int32).reshape(N, 1)

    grid = (N // BLOCK_R,)
    out_shapes = (
        jax.ShapeDtypeStruct((N, 37), jnp.float32),
        jax.ShapeDtypeStruct((N, K), jnp.int32),
        jax.ShapeDtypeStruct((N, K), jnp.int32),
        jax.ShapeDtypeStruct((N, K), jnp.float32),
        jax.ShapeDtypeStruct((N, K), jnp.float32),
        jax.ShapeDtypeStruct((N, K), jnp.float32),
        jax.ShapeDtypeStruct((N, K), jnp.float32),
    )
    in_specs = [
        pl.BlockSpec((BLOCK_R, SEQ * DIM), lambda i: (i, 0)),
        pl.BlockSpec((8, N), lambda i: (0, 0)),
        pl.BlockSpec((BLOCK_R, 1), lambda i: (i, 0)),
        pl.BlockSpec((NTYPES, EMB), lambda i: (0, 0)),
    ]
    out_specs = (
        pl.BlockSpec((BLOCK_R, 37), lambda i: (i, 0)),
        pl.BlockSpec((BLOCK_R, K), lambda i: (i, 0)),
        pl.BlockSpec((BLOCK_R, K), lambda i: (i, 0)),
        pl.BlockSpec((BLOCK_R, K), lambda i: (i, 0)),
        pl.BlockSpec((BLOCK_R, K), lambda i: (i, 0)),
        pl.BlockSpec((BLOCK_R, K), lambda i: (i, 0)),
        pl.BlockSpec((BLOCK_R, K), lambda i: (i, 0)),
    )
    node, send, recv, dx, dy, dz, dd = pl.pallas_call(
        _knn_kernel,
        grid=grid,
        in_specs=in_specs,
        out_specs=out_specs,
        out_shape=out_shapes,
    )(psf, post, types, embed_table)

    edge_index = jnp.stack([send.reshape(-1), recv.reshape(-1)], axis=0)
    edge_features = jnp.stack([dx, dy, dz, dd], axis=-1).reshape(N * K, 4)
    return node, edge_index, edge_features
